# Initial kernel scaffold; baseline (speedup 1.0000x reference)
import jax
import jax.numpy as jnp
from jax.experimental import pallas as pl


def kernel(embedding, initial_preds, candidate_cells, candidate_probs, protos, proto_latlon, temperature, geo_scaling):
    raise NotImplementedError("write your pallas kernel here")


# speedup vs baseline: 2.1491x; 2.1491x over previous
import jax
import jax.numpy as jnp
from jax import lax
from jax.experimental import pallas as pl
from jax.experimental.pallas import tpu as pltpu

_B = 256
_D = 768
_G = 1024
_P = 32
_T = 5
_C = 128
_STEPS = _G // _C


def _body(emb_ref, protos_ref, cells_ref, probs_ref, latlon_ref, temp_ref,
          geo_ref, llh_ref, comb_ref, cell_ref, sim_sc, idx_sc):
    i = pl.program_id(0)
    emb = emb_ref[:, :]
    geo = geo_ref[0, 0]
    e2 = jnp.sum(emb * emb, axis=1, keepdims=True)
    best = None
    bidx = None
    for p in range(_P):
        pr = protos_ref[:, p, :]
        dotp = lax.dot_general(emb, pr, (((1,), (1,)), ((), ())),
                               preferred_element_type=jnp.float32)
        p2 = jnp.sum(pr * pr, axis=1)[None, :]
        dist = e2 + p2 - 2.0 * dotp
        sim = -dist / geo
        if p == 0:
            best = sim
            bidx = jnp.zeros((_B, _C), jnp.int32)
        else:
            m = sim > best
            best = jnp.where(m, sim, best)
            bidx = jnp.where(m, p, bidx)
    sim_sc[:, pl.ds(i * _C, _C)] = best
    idx_sc[:, pl.ds(i * _C, _C)] = bidx

    @pl.when(i == _STEPS - 1)
    def _final():
        temp = temp_ref[0, 0]
        sim_all = sim_sc[:, :]
        idx_all = idx_sc[:, :]
        giota = lax.broadcasted_iota(jnp.int32, (_B, _G), 1)
        bs, bi, cs = [], [], []
        for t in range(_T):
            ct = cells_ref[:, t:t + 1]
            mask = giota == ct
            bs.append(jnp.max(jnp.where(mask, sim_all, -jnp.inf),
                              axis=1, keepdims=True))
            bi.append(jnp.max(jnp.where(mask, idx_all, 0),
                              axis=1, keepdims=True))
            cs.append(ct)
        x = [b / temp for b in bs]
        xm = x[0]
        for t in range(1, _T):
            xm = jnp.maximum(xm, x[t])
        ex = [jnp.exp(xt - xm) for xt in x]
        es = ex[0]
        for t in range(1, _T):
            es = es + ex[t]
        scores = [e / es for e in ex]
        cb = [scores[t] * probs_ref[:, t:t + 1] for t in range(_T)]
        den = cb[0]
        for t in range(1, _T):
            den = den + cb[t]
        den = den + 1e-9
        cb = [c / den for c in cb]
        for t in range(_T):
            comb_ref[:, t:t + 1] = cb[t]
        bv, cellv, protov = cb[0], cs[0], bi[0]
        for t in range(1, _T):
            mm = cb[t] > bv
            bv = jnp.where(mm, cb[t], bv)
            cellv = jnp.where(mm, cs[t], cellv)
            protov = jnp.where(mm, bi[t], protov)
        cell_ref[:, :] = cellv
        onehot = (giota == cellv).astype(jnp.float32)
        rows = lax.dot_general(onehot, latlon_ref[:, :],
                               (((1,), (0,)), ((), ())),
                               precision=lax.Precision.HIGHEST,
                               preferred_element_type=jnp.float32)
        pi = lax.broadcasted_iota(jnp.int32, (_B, 2 * _P), 1)
        lat = jnp.sum(jnp.where(pi == 2 * protov, rows, 0.0),
                      axis=1, keepdims=True)
        lon = jnp.sum(jnp.where(pi == 2 * protov + 1, rows, 0.0),
                      axis=1, keepdims=True)
        llh_ref[:, 0:1] = lat
        llh_ref[:, 1:2] = lon


def kernel(embedding, initial_preds, candidate_cells, candidate_probs,
           protos, proto_latlon, temperature, geo_scaling):
    if embedding.ndim == 3:
        embedding = embedding.mean(axis=1)
    emb = embedding.astype(jnp.float32)
    cells = candidate_cells[:, :_T]
    probs = candidate_probs[:, :_T]
    latlon2 = proto_latlon.reshape(_G, 2 * _P)
    temp = jnp.reshape(temperature, (1, 1)).astype(jnp.float32)
    geo = jnp.reshape(geo_scaling, (1, 1)).astype(jnp.float32)

    llh, comb, cellc = pl.pallas_call(
        _body,
        grid=(_STEPS,),
        in_specs=[
            pl.BlockSpec((_B, _D), lambda i: (0, 0)),
            pl.BlockSpec((_C, _P, _D), lambda i: (i, 0, 0)),
            pl.BlockSpec((_B, _T), lambda i: (0, 0)),
            pl.BlockSpec((_B, _T), lambda i: (0, 0)),
            pl.BlockSpec((_G, 2 * _P), lambda i: (0, 0)),
            pl.BlockSpec((1, 1), lambda i: (0, 0)),
            pl.BlockSpec((1, 1), lambda i: (0, 0)),
        ],
        out_specs=[
            pl.BlockSpec((_B, 2), lambda i: (0, 0)),
            pl.BlockSpec((_B, _T), lambda i: (0, 0)),
            pl.BlockSpec((_B, 1), lambda i: (0, 0)),
        ],
        out_shape=[
            jax.ShapeDtypeStruct((_B, 2), jnp.float32),
            jax.ShapeDtypeStruct((_B, _T), jnp.float32),
            jax.ShapeDtypeStruct((_B, 1), jnp.int32),
        ],
        scratch_shapes=[
            pltpu.VMEM((_B, _G), jnp.float32),
            pltpu.VMEM((_B, _G), jnp.int32),
        ],
        compiler_params=pltpu.CompilerParams(
            dimension_semantics=("arbitrary",),
        ),
    )(emb, protos, cells, probs, latlon2, temp, geo)
    return llh, comb, cellc[:, 0]


# speedup vs baseline: 2.5617x; 1.1920x over previous
import jax
import jax.numpy as jnp
from jax import lax
from jax.experimental import pallas as pl
from jax.experimental.pallas import tpu as pltpu

_B = 256
_D = 768
_G = 1024
_P = 32
_T = 5
_C = 128
_STEPS = _G // _C


def _body(emb_ref, embt_ref, protos_ref, cells_ref, probs_ref, latlon_ref,
          temp_ref, geo_ref, llh_ref, comb_ref, cell_ref, sim_sc, idx_sc):
    i = pl.program_id(0)
    embt = embt_ref[:, :]
    geo = geo_ref[0, 0]
    emb = emb_ref[:, :]
    e2c = jnp.sum(emb * emb, axis=1, keepdims=True)
    e2 = jnp.transpose(e2c)
    best = None
    bidx = None
    for p in range(_P):
        pr = protos_ref[:, p, :]
        dotp = lax.dot_general(pr, embt, (((1,), (0,)), ((), ())),
                               preferred_element_type=jnp.float32)
        p2 = jnp.sum(pr * pr, axis=1, keepdims=True)
        dist = e2 + p2 - 2.0 * dotp
        sim = -dist / geo
        if p == 0:
            best = sim
            bidx = jnp.zeros((_C, _B), jnp.int32)
        else:
            m = sim > best
            best = jnp.where(m, sim, best)
            bidx = jnp.where(m, p, bidx)
    sim_sc[pl.ds(i * _C, _C), :] = best
    idx_sc[pl.ds(i * _C, _C), :] = bidx

    @pl.when(i == _STEPS - 1)
    def _final():
        temp = temp_ref[0, 0]
        sim_all = sim_sc[:, :]
        idx_all = idx_sc[:, :]
        giota = lax.broadcasted_iota(jnp.int32, (_G, _B), 0)
        bs, bi, cs = [], [], []
        for t in range(_T):
            ct = cells_ref[t:t + 1, :]
            mask = giota == ct
            bs.append(jnp.max(jnp.where(mask, sim_all, -jnp.inf),
                              axis=0, keepdims=True))
            bi.append(jnp.max(jnp.where(mask, idx_all, 0),
                              axis=0, keepdims=True))
            cs.append(ct)
        x = [b / temp for b in bs]
        xm = x[0]
        for t in range(1, _T):
            xm = jnp.maximum(xm, x[t])
        ex = [jnp.exp(xt - xm) for xt in x]
        es = ex[0]
        for t in range(1, _T):
            es = es + ex[t]
        scores = [e / es for e in ex]
        cb = [scores[t] * probs_ref[t:t + 1, :] for t in range(_T)]
        den = cb[0]
        for t in range(1, _T):
            den = den + cb[t]
        den = den + 1e-9
        cb = [c / den for c in cb]
        for t in range(_T):
            comb_ref[t:t + 1, :] = cb[t]
        bv, cellv, protov = cb[0], cs[0], bi[0]
        for t in range(1, _T):
            mm = cb[t] > bv
            bv = jnp.where(mm, cb[t], bv)
            cellv = jnp.where(mm, cs[t], cellv)
            protov = jnp.where(mm, bi[t], protov)
        cell_ref[:, :] = cellv
        onehot = (giota == cellv).astype(jnp.float32)
        rows = lax.dot_general(latlon_ref[:, :], onehot,
                               (((1,), (0,)), ((), ())),
                               precision=lax.Precision.HIGHEST,
                               preferred_element_type=jnp.float32)
        pi = lax.broadcasted_iota(jnp.int32, (2 * _P, _B), 0)
        lat = jnp.sum(jnp.where(pi == 2 * protov, rows, 0.0),
                      axis=0, keepdims=True)
        lon = jnp.sum(jnp.where(pi == 2 * protov + 1, rows, 0.0),
                      axis=0, keepdims=True)
        llh_ref[0:1, :] = lat
        llh_ref[1:2, :] = lon


def kernel(embedding, initial_preds, candidate_cells, candidate_probs,
           protos, proto_latlon, temperature, geo_scaling):
    if embedding.ndim == 3:
        embedding = embedding.mean(axis=1)
    emb = embedding.astype(jnp.float32)
    embt = emb.T
    cellst = candidate_cells[:, :_T].T
    cellst = jnp.pad(cellst, ((0, 3), (0, 0)))
    probst = candidate_probs[:, :_T].T
    probst = jnp.pad(probst, ((0, 3), (0, 0)))
    latlont = proto_latlon.reshape(_G, 2 * _P).T
    temp = jnp.reshape(temperature, (1, 1)).astype(jnp.float32)
    geo = jnp.reshape(geo_scaling, (1, 1)).astype(jnp.float32)

    llh, comb, cellc = pl.pallas_call(
        _body,
        grid=(_STEPS,),
        in_specs=[
            pl.BlockSpec((_B, _D), lambda i: (0, 0)),
            pl.BlockSpec((_D, _B), lambda i: (0, 0)),
            pl.BlockSpec((_C, _P, _D), lambda i: (i, 0, 0)),
            pl.BlockSpec((8, _B), lambda i: (0, 0)),
            pl.BlockSpec((8, _B), lambda i: (0, 0)),
            pl.BlockSpec((2 * _P, _G), lambda i: (0, 0)),
            pl.BlockSpec((1, 1), lambda i: (0, 0)),
            pl.BlockSpec((1, 1), lambda i: (0, 0)),
        ],
        out_specs=[
            pl.BlockSpec((2, _B), lambda i: (0, 0)),
            pl.BlockSpec((8, _B), lambda i: (0, 0)),
            pl.BlockSpec((1, _B), lambda i: (0, 0)),
        ],
        out_shape=[
            jax.ShapeDtypeStruct((2, _B), jnp.float32),
            jax.ShapeDtypeStruct((8, _B), jnp.float32),
            jax.ShapeDtypeStruct((1, _B), jnp.int32),
        ],
        scratch_shapes=[
            pltpu.VMEM((_G, _B), jnp.float32),
            pltpu.VMEM((_G, _B), jnp.int32),
        ],
        compiler_params=pltpu.CompilerParams(
            dimension_semantics=("arbitrary",),
        ),
    )(emb, embt, protos, cellst, probst, latlont, temp, geo)
    return llh.T, comb[:_T, :].T, cellc[0, :]


# speedup vs baseline: 4.7515x; 1.8548x over previous
import jax
import jax.numpy as jnp
from jax import lax
from jax.experimental import pallas as pl
from jax.experimental.pallas import tpu as pltpu

_B = 256
_D = 768
_G = 1024
_P = 32
_T = 5
_C = 128
_STEPS = _G // _C
_M = 512
_RPS = _C * _P


def _body(emb_ref, embt_ref, protos_ref, cells_ref, probs_ref, latlon_ref,
          temp_ref, geo_ref, llh_ref, comb_ref, cell_ref, sim_sc, idx_sc):
    i = pl.program_id(0)
    embt = embt_ref[:, :]
    geo = geo_ref[0, 0]
    emb = emb_ref[:, :]
    e2c = jnp.sum(emb * emb, axis=1, keepdims=True)
    e2 = jnp.transpose(e2c)
    piota = lax.broadcasted_iota(jnp.int32, (_M // _P, _P, _B), 1)
    for m in range(_RPS // _M):
        rows = protos_ref[pl.ds(m * _M, _M), :]
        dotp = lax.dot_general(rows, embt, (((1,), (0,)), ((), ())),
                               preferred_element_type=jnp.float32)
        p2 = jnp.sum(rows * rows, axis=1, keepdims=True)
        dist = e2 + p2 - 2.0 * dotp
        sim = -dist / geo
        r3 = sim.reshape(_M // _P, _P, _B)
        gmax = jnp.max(r3, axis=1)
        eq = r3 == gmax[:, None, :]
        gidx = jnp.min(jnp.where(eq, piota, _P), axis=1)
        base = i * _C + m * (_M // _P)
        sim_sc[pl.ds(base, _M // _P), :] = gmax
        idx_sc[pl.ds(base, _M // _P), :] = gidx

    @pl.when(i == _STEPS - 1)
    def _final():
        temp = temp_ref[0, 0]
        sim_all = sim_sc[:, :]
        idx_all = idx_sc[:, :]
        giota = lax.broadcasted_iota(jnp.int32, (_G, _B), 0)
        bs, bi, cs = [], [], []
        for t in range(_T):
            ct = cells_ref[t:t + 1, :]
            mask = giota == ct
            bs.append(jnp.max(jnp.where(mask, sim_all, -jnp.inf),
                              axis=0, keepdims=True))
            bi.append(jnp.max(jnp.where(mask, idx_all, 0),
                              axis=0, keepdims=True))
            cs.append(ct)
        x = [b / temp for b in bs]
        xm = x[0]
        for t in range(1, _T):
            xm = jnp.maximum(xm, x[t])
        ex = [jnp.exp(xt - xm) for xt in x]
        es = ex[0]
        for t in range(1, _T):
            es = es + ex[t]
        scores = [e / es for e in ex]
        cb = [scores[t] * probs_ref[t:t + 1, :] for t in range(_T)]
        den = cb[0]
        for t in range(1, _T):
            den = den + cb[t]
        den = den + 1e-9
        cb = [c / den for c in cb]
        for t in range(_T):
            comb_ref[t:t + 1, :] = cb[t]
        bv, cellv, protov = cb[0], cs[0], bi[0]
        for t in range(1, _T):
            mm = cb[t] > bv
            bv = jnp.where(mm, cb[t], bv)
            cellv = jnp.where(mm, cs[t], cellv)
            protov = jnp.where(mm, bi[t], protov)
        cell_ref[:, :] = cellv
        onehot = (giota == cellv).astype(jnp.float32)
        rows = lax.dot_general(latlon_ref[:, :], onehot,
                               (((1,), (0,)), ((), ())),
                               precision=lax.Precision.HIGHEST,
                               preferred_element_type=jnp.float32)
        pi = lax.broadcasted_iota(jnp.int32, (2 * _P, _B), 0)
        lat = jnp.sum(jnp.where(pi == 2 * protov, rows, 0.0),
                      axis=0, keepdims=True)
        lon = jnp.sum(jnp.where(pi == 2 * protov + 1, rows, 0.0),
                      axis=0, keepdims=True)
        llh_ref[0:1, :] = lat
        llh_ref[1:2, :] = lon


def kernel(embedding, initial_preds, candidate_cells, candidate_probs,
           protos, proto_latlon, temperature, geo_scaling):
    if embedding.ndim == 3:
        embedding = embedding.mean(axis=1)
    emb = embedding.astype(jnp.float32)
    embt = emb.T
    protos_flat = protos.reshape(_G * _P, _D)
    cellst = candidate_cells[:, :_T].T
    cellst = jnp.pad(cellst, ((0, 3), (0, 0)))
    probst = candidate_probs[:, :_T].T
    probst = jnp.pad(probst, ((0, 3), (0, 0)))
    latlont = proto_latlon.reshape(_G, 2 * _P).T
    temp = jnp.reshape(temperature, (1, 1)).astype(jnp.float32)
    geo = jnp.reshape(geo_scaling, (1, 1)).astype(jnp.float32)

    llh, comb, cellc = pl.pallas_call(
        _body,
        grid=(_STEPS,),
        in_specs=[
            pl.BlockSpec((_B, _D), lambda i: (0, 0)),
            pl.BlockSpec((_D, _B), lambda i: (0, 0)),
            pl.BlockSpec((_RPS, _D), lambda i: (i, 0)),
            pl.BlockSpec((8, _B), lambda i: (0, 0)),
            pl.BlockSpec((8, _B), lambda i: (0, 0)),
            pl.BlockSpec((2 * _P, _G), lambda i: (0, 0)),
            pl.BlockSpec((1, 1), lambda i: (0, 0)),
            pl.BlockSpec((1, 1), lambda i: (0, 0)),
        ],
        out_specs=[
            pl.BlockSpec((2, _B), lambda i: (0, 0)),
            pl.BlockSpec((8, _B), lambda i: (0, 0)),
            pl.BlockSpec((1, _B), lambda i: (0, 0)),
        ],
        out_shape=[
            jax.ShapeDtypeStruct((2, _B), jnp.float32),
            jax.ShapeDtypeStruct((8, _B), jnp.float32),
            jax.ShapeDtypeStruct((1, _B), jnp.int32),
        ],
        scratch_shapes=[
            pltpu.VMEM((_G, _B), jnp.float32),
            pltpu.VMEM((_G, _B), jnp.int32),
        ],
        compiler_params=pltpu.CompilerParams(
            dimension_semantics=("arbitrary",),
        ),
    )(emb, embt, protos_flat, cellst, probst, latlont, temp, geo)
    return llh.T, comb[:_T, :].T, cellc[0, :]
